# baseline (speedup 1.0000x reference)
import functools

import jax
import jax.numpy as jnp
from jax import lax
from jax.experimental import pallas as pl
from jax.experimental.pallas import tpu as pltpu
from jax.experimental.pallas import tpu_sc as plsc

_IGNORE = 255
_NCLS = 19
_M = 8
_D = 10
_K = 26
_NSEG = _M * _M * _K
_SEGP = 1792
_TBL_ONE = _NCLS * _SEGP
_TBL = 3 * _TBL_ONE
_B, _W, _H = 4, 512, 512
_N = _B * _W * _H
_NW = 32
_CHUNK = _N // _NW
_SUB = 2048
_NSUB = _CHUNK // _SUB
_MG = 14592
_MROUNDS = _TBL // _MG
_SLICE = _MG // 16


def _f64_to_f32(x):
    if x.dtype != jnp.float64:
        return x.astype(jnp.float32)
    b = lax.bitcast_convert_type(x, jnp.uint32)
    return lax.bitcast_convert_type(b[..., 1], jnp.float32)


def _pack_body(pred_ref, hm_ref, depth_ref, packed_ref):
    j = pl.program_id(1)
    rows = lax.broadcasted_iota(jnp.int32, (256, 512), 0) + j * 256
    cols = lax.broadcasted_iota(jnp.int32, (256, 512), 1)
    rb = rows // 64
    cb = cols // 64
    row_in = (rows - rb * 64) < 63
    col_in = (cols - cb * 64) < 63
    blk = rb * _M + cb

    d = depth_ref[0]
    hm = hm_ref[0]
    pr = pred_ref[0]

    kb = jnp.clip(jnp.floor(d / float(_D)).astype(jnp.int32), 0, _K - 1)
    kbf = kb.astype(jnp.float32)
    bin_valid = (d > kbf * _D) & (d < kbf * _D + (_D - 1))
    valid = hm != _IGNORE
    overall = row_in & col_in & valid & bin_valid
    seg = jnp.where(overall, blk * _K + kb, _NSEG)

    hmc = jnp.clip(hm, 0, _NCLS - 1)
    prc = jnp.clip(pr, 0, _NCLS - 1)
    il = hmc * _SEGP + seg
    ip = prc * _SEGP + seg
    packed_ref[0] = il | (ip << 16)


_pack = pl.pallas_call(
    _pack_body,
    grid=(_B, 2),
    in_specs=[
        pl.BlockSpec((1, 256, 512), lambda i, j: (i, j, 0)),
        pl.BlockSpec((1, 256, 512), lambda i, j: (i, j, 0)),
        pl.BlockSpec((1, 256, 512), lambda i, j: (i, j, 0)),
    ],
    out_specs=pl.BlockSpec((1, 256, 512), lambda i, j: (i, j, 0)),
    out_shape=jax.ShapeDtypeStruct((_B, _W, _H), jnp.int32),
)


def _sums_body(depth_ref, pd_ref, loss_ref, hm_ref, sums_ref):
    i = pl.program_id(0)
    j = pl.program_id(1)
    d = depth_ref[0]
    pd = pd_ref[0]
    l1 = loss_ref[0]
    hm = hm_ref[0]

    diff = jnp.log(pd * 255.0 + 1.0) - jnp.log(d + 1.0)
    s0 = jnp.sum(diff * diff)
    s1 = jnp.sum(diff)
    s2 = jnp.sum(l1 * jnp.abs(diff))
    s3 = jnp.sum((hm != _IGNORE).astype(jnp.float32))

    @pl.when((i == 0) & (j == 0))
    def _():
        sums_ref[0] = 0.0
        sums_ref[1] = 0.0
        sums_ref[2] = 0.0
        sums_ref[3] = 0.0

    sums_ref[0] += s0
    sums_ref[1] += s1
    sums_ref[2] += s2
    sums_ref[3] += s3


_sums = pl.pallas_call(
    _sums_body,
    grid=(_B, 2),
    in_specs=[
        pl.BlockSpec((1, 256, 512), lambda i, j: (i, j, 0)),
        pl.BlockSpec((1, 256, 512), lambda i, j: (i, j, 0)),
        pl.BlockSpec((1, 256, 512), lambda i, j: (i, j, 0)),
        pl.BlockSpec((1, 256, 512), lambda i, j: (i, j, 0)),
    ],
    out_specs=pl.BlockSpec(memory_space=pltpu.SMEM, block_shape=(4,), index_map=lambda i, j: (0,)),
    out_shape=jax.ShapeDtypeStruct((4,), jnp.float32),
)


def _sc_body(packed_hbm, loss_hbm, out_hbm, tbl, pk0, pk1, ls0, ls1, sem0, sem1):
    core = lax.axis_index("c")
    sid = lax.axis_index("s")
    wid = core * 16 + sid
    base = wid * _CHUNK

    zf = jnp.zeros((16,), jnp.float32)
    onef = jnp.ones((16,), jnp.float32)
    mask16 = jnp.full((16,), 0xFFFF, jnp.int32)
    shift16 = jnp.full((16,), 16, jnp.int32)
    off_p = jnp.full((16,), _TBL_ONE, jnp.int32)
    off_l = jnp.full((16,), 2 * _TBL_ONE, jnp.int32)

    def zbody(i, c):
        tbl[pl.ds(i * 64, 16)] = zf
        tbl[pl.ds(i * 64 + 16, 16)] = zf
        tbl[pl.ds(i * 64 + 32, 16)] = zf
        tbl[pl.ds(i * 64 + 48, 16)] = zf
        return c

    lax.fori_loop(0, _TBL // 64, zbody, 0, unroll=4)

    bufs = ((pk0, ls0, sem0), (pk1, ls1, sem1))

    def start(k, b):
        pk, ls, sem = bufs[b]
        h0 = pltpu.async_copy(packed_hbm.at[pl.ds(base + k * _SUB, _SUB)], pk, sem)
        h1 = pltpu.async_copy(loss_hbm.at[pl.ds(base + k * _SUB, _SUB)], ls, sem)
        return (h0, h1)

    def process(b):
        pk, ls, _ = bufs[b]

        def pbody(i, c):
            v = pk[pl.ds(i * 16, 16)]
            il = v & mask16
            ip = lax.shift_right_logical(v, shift16)
            lv = ls[pl.ds(i * 16, 16)]
            plsc.addupdate_scatter(tbl, [il], onef)
            plsc.addupdate_scatter(tbl, [ip + off_p], onef)
            plsc.addupdate_scatter(tbl, [il + off_l], lv)
            return c

        lax.fori_loop(0, _SUB // 16, pbody, 0, unroll=4)

    pending = start(0, 0)
    for k in range(_NSUB):
        b = k & 1
        cur = pending
        if k + 1 < _NSUB:
            pending = start(k + 1, 1 - b)
        cur[0].wait()
        cur[1].wait()
        process(b)

    pltpu.sync_copy(tbl, out_hbm.at[pl.ds(wid * _TBL, _TBL)])


_sc_scatter = functools.partial(
    pl.kernel,
    out_type=jax.ShapeDtypeStruct((_NW * _TBL,), jnp.float32),
    mesh=plsc.VectorSubcoreMesh(
        core_axis_name="c", subcore_axis_name="s", num_cores=2, num_subcores=16
    ),
    scratch_types=[
        pltpu.VMEM((_TBL,), jnp.float32),
        pltpu.VMEM((_SUB,), jnp.int32),
        pltpu.VMEM((_SUB,), jnp.int32),
        pltpu.VMEM((_SUB,), jnp.float32),
        pltpu.VMEM((_SUB,), jnp.float32),
        pltpu.SemaphoreType.DMA,
        pltpu.SemaphoreType.DMA,
    ],
    compiler_params=pltpu.CompilerParams(needs_layout_passes=False),
)(_sc_body)


def _finish_body(sums_ref, tab_ref, out_ref):
    t = jnp.sum(tab_ref[...], axis=0)
    cl = t[0:_NCLS, :]
    cp = t[_NCLS : 2 * _NCLS, :]
    lt = t[2 * _NCLS : 3 * _NCLS, :]

    counts = jnp.sum(cl, axis=0, keepdims=True)
    lsum = jnp.sum(lt, axis=0, keepdims=True)
    sym = jnp.sum(((cl > 0.0) != (cp > 0.0)).astype(jnp.float32), axis=0, keepdims=True)

    colid = lax.broadcasted_iota(jnp.int32, (1, _SEGP), 1)
    segvalid = colid < _NSEG
    active = (counts > 0.0) & segvalid
    mean = lsum / jnp.maximum(counts, 1.0)
    times = jnp.sum(active.astype(jnp.float32))
    loss3 = jnp.sum(jnp.where(active, sym * mean, 0.0)) / (times + 0.001)

    nf = float(_N)
    data_loss = sums_ref[0] / nf - 0.5 * (sums_ref[1] * sums_ref[1]) / (nf * nf)
    lr = sums_ref[2] / sums_ref[3]
    out_ref[0] = data_loss
    out_ref[1] = lr
    out_ref[2] = loss3


_finish = pl.pallas_call(
    _finish_body,
    in_specs=[
        pl.BlockSpec(memory_space=pltpu.SMEM),
        pl.BlockSpec((_NW, 3 * _NCLS, _SEGP), lambda: (0, 0, 0)),
    ],
    out_specs=pl.BlockSpec(memory_space=pltpu.SMEM),
    out_shape=jax.ShapeDtypeStruct((3,), jnp.float32),
)


def _run(pred, heatmaps, depth, loss_1, pred_depth):
    pred32 = pred.astype(jnp.int32)
    hm32 = heatmaps.astype(jnp.int32)
    d32 = _f64_to_f32(depth)
    l32 = _f64_to_f32(loss_1)
    packed = _pack(pred32, hm32, d32)
    tabs = _sc_scatter(packed.reshape(_N), l32.reshape(_N))
    pd32 = _f64_to_f32(pred_depth)
    sums = _sums(d32, pd32, l32, hm32)
    return _finish(sums, tabs.reshape(_NW, 3 * _NCLS, _SEGP))


def kernel(pred, heatmaps, depth, loss_1, pred_depth, epoch):
    odt = jnp.result_type(depth.dtype, jnp.float32)
    with jax.enable_x64(False):
        out = _run(pred, heatmaps, depth, loss_1, pred_depth)
    out = out.astype(odt)
    return out[0], out[1], out[2]


# speedup vs baseline: 2.1222x; 2.1222x over previous
import functools

import jax
import jax.numpy as jnp
from jax import lax
from jax.experimental import pallas as pl
from jax.experimental.pallas import tpu as pltpu
from jax.experimental.pallas import tpu_sc as plsc

_IGNORE = 255
_NCLS = 19
_M = 8
_D = 10
_K = 26
_NSEG = _M * _M * _K
_SEGP = 1792
_TBL_ONE = _NCLS * _SEGP
_TBL = 3 * _TBL_ONE
_B, _W, _H = 4, 512, 512
_N = _B * _W * _H
_NW = 32
_CHUNK = _N // _NW
_SUB = 2048
_NSUB = _CHUNK // _SUB
_MG = 14592
_MROUNDS = _TBL // _MG
_SLICE = _MG // 16


def _f64_to_f32(x):
    return x.astype(jnp.float32)


def _pack_body(pred_ref, hm_ref, depth_ref, packed_ref):
    j = pl.program_id(1)
    rows = lax.broadcasted_iota(jnp.int32, (256, 512), 0) + j * 256
    cols = lax.broadcasted_iota(jnp.int32, (256, 512), 1)
    rb = rows // 64
    cb = cols // 64
    row_in = (rows - rb * 64) < 63
    col_in = (cols - cb * 64) < 63
    blk = rb * _M + cb

    d = depth_ref[0]
    hm = hm_ref[0]
    pr = pred_ref[0]

    kb = jnp.clip(jnp.floor(d / float(_D)).astype(jnp.int32), 0, _K - 1)
    kbf = kb.astype(jnp.float32)
    bin_valid = (d > kbf * _D) & (d < kbf * _D + (_D - 1))
    valid = hm != _IGNORE
    overall = row_in & col_in & valid & bin_valid
    seg = jnp.where(overall, blk * _K + kb, _NSEG)

    hmc = jnp.clip(hm, 0, _NCLS - 1)
    prc = jnp.clip(pr, 0, _NCLS - 1)
    il = hmc * _SEGP + seg
    ip = prc * _SEGP + seg
    packed_ref[0] = il | (ip << 16)


_pack = pl.pallas_call(
    _pack_body,
    grid=(_B, 2),
    in_specs=[
        pl.BlockSpec((1, 256, 512), lambda i, j: (i, j, 0)),
        pl.BlockSpec((1, 256, 512), lambda i, j: (i, j, 0)),
        pl.BlockSpec((1, 256, 512), lambda i, j: (i, j, 0)),
    ],
    out_specs=pl.BlockSpec((1, 256, 512), lambda i, j: (i, j, 0)),
    out_shape=jax.ShapeDtypeStruct((_B, _W, _H), jnp.int32),
)


def _sums_body(depth_ref, pd_ref, loss_ref, hm_ref, sums_ref):
    i = pl.program_id(0)
    j = pl.program_id(1)
    d = depth_ref[0]
    pd = pd_ref[0]
    l1 = loss_ref[0]
    hm = hm_ref[0]

    diff = jnp.log(pd * 255.0 + 1.0) - jnp.log(d + 1.0)
    s0 = jnp.sum(diff * diff)
    s1 = jnp.sum(diff)
    s2 = jnp.sum(l1 * jnp.abs(diff))
    s3 = jnp.sum((hm != _IGNORE).astype(jnp.float32))

    @pl.when((i == 0) & (j == 0))
    def _():
        sums_ref[0] = 0.0
        sums_ref[1] = 0.0
        sums_ref[2] = 0.0
        sums_ref[3] = 0.0

    sums_ref[0] += s0
    sums_ref[1] += s1
    sums_ref[2] += s2
    sums_ref[3] += s3


_sums = pl.pallas_call(
    _sums_body,
    grid=(_B, 2),
    in_specs=[
        pl.BlockSpec((1, 256, 512), lambda i, j: (i, j, 0)),
        pl.BlockSpec((1, 256, 512), lambda i, j: (i, j, 0)),
        pl.BlockSpec((1, 256, 512), lambda i, j: (i, j, 0)),
        pl.BlockSpec((1, 256, 512), lambda i, j: (i, j, 0)),
    ],
    out_specs=pl.BlockSpec(memory_space=pltpu.SMEM, block_shape=(4,), index_map=lambda i, j: (0,)),
    out_shape=jax.ShapeDtypeStruct((4,), jnp.float32),
)


def _sc_body(packed_hbm, loss_hbm, out_hbm, tbl, pk0, pk1, ls0, ls1, sem0, sem1):
    core = lax.axis_index("c")
    sid = lax.axis_index("s")
    wid = core * 16 + sid
    base = wid * _CHUNK

    zf = jnp.zeros((16,), jnp.float32)
    onef = jnp.ones((16,), jnp.float32)
    mask16 = jnp.full((16,), 0xFFFF, jnp.int32)
    shift16 = jnp.full((16,), 16, jnp.int32)
    off_p = jnp.full((16,), _TBL_ONE, jnp.int32)
    off_l = jnp.full((16,), 2 * _TBL_ONE, jnp.int32)

    def zbody(i, c):
        tbl[pl.ds(i * 64, 16)] = zf
        tbl[pl.ds(i * 64 + 16, 16)] = zf
        tbl[pl.ds(i * 64 + 32, 16)] = zf
        tbl[pl.ds(i * 64 + 48, 16)] = zf
        return c

    lax.fori_loop(0, _TBL // 64, zbody, 0, unroll=4)

    bufs = ((pk0, ls0, sem0), (pk1, ls1, sem1))

    def start(k, b):
        pk, ls, sem = bufs[b]
        h0 = pltpu.async_copy(packed_hbm.at[pl.ds(base + k * _SUB, _SUB)], pk, sem)
        h1 = pltpu.async_copy(loss_hbm.at[pl.ds(base + k * _SUB, _SUB)], ls, sem)
        return (h0, h1)

    def process(b):
        pk, ls, _ = bufs[b]

        def pbody(i, c):
            v = pk[pl.ds(i * 16, 16)]
            il = v & mask16
            ip = lax.shift_right_logical(v, shift16)
            lv = ls[pl.ds(i * 16, 16)]
            plsc.addupdate_scatter(tbl, [il], onef)
            plsc.addupdate_scatter(tbl, [ip + off_p], onef)
            plsc.addupdate_scatter(tbl, [il + off_l], lv)
            return c

        lax.fori_loop(0, _SUB // 16, pbody, 0, unroll=4)

    pending = start(0, 0)
    for k in range(_NSUB):
        b = k & 1
        cur = pending
        if k + 1 < _NSUB:
            pending = start(k + 1, 1 - b)
        cur[0].wait()
        cur[1].wait()
        process(b)

    pltpu.sync_copy(tbl, out_hbm.at[pl.ds(wid * _TBL, _TBL)])


_sc_scatter = functools.partial(
    pl.kernel,
    out_type=jax.ShapeDtypeStruct((_NW * _TBL,), jnp.float32),
    mesh=plsc.VectorSubcoreMesh(
        core_axis_name="c", subcore_axis_name="s", num_cores=2, num_subcores=16
    ),
    scratch_types=[
        pltpu.VMEM((_TBL,), jnp.float32),
        pltpu.VMEM((_SUB,), jnp.int32),
        pltpu.VMEM((_SUB,), jnp.int32),
        pltpu.VMEM((_SUB,), jnp.float32),
        pltpu.VMEM((_SUB,), jnp.float32),
        pltpu.SemaphoreType.DMA,
        pltpu.SemaphoreType.DMA,
    ],
    compiler_params=pltpu.CompilerParams(needs_layout_passes=False),
)(_sc_body)


def _finish_body(sums_ref, tab_ref, out_ref):
    t = jnp.sum(tab_ref[...], axis=0)
    cl = t[0:_NCLS, :]
    cp = t[_NCLS : 2 * _NCLS, :]
    lt = t[2 * _NCLS : 3 * _NCLS, :]

    counts = jnp.sum(cl, axis=0, keepdims=True)
    lsum = jnp.sum(lt, axis=0, keepdims=True)
    sym = jnp.sum(((cl > 0.0) != (cp > 0.0)).astype(jnp.float32), axis=0, keepdims=True)

    colid = lax.broadcasted_iota(jnp.int32, (1, _SEGP), 1)
    segvalid = colid < _NSEG
    active = (counts > 0.0) & segvalid
    mean = lsum / jnp.maximum(counts, 1.0)
    times = jnp.sum(active.astype(jnp.float32))
    loss3 = jnp.sum(jnp.where(active, sym * mean, 0.0)) / (times + 0.001)

    nf = float(_N)
    data_loss = sums_ref[0] / nf - 0.5 * (sums_ref[1] * sums_ref[1]) / (nf * nf)
    lr = sums_ref[2] / sums_ref[3]
    out_ref[0] = data_loss
    out_ref[1] = lr
    out_ref[2] = loss3


_finish = pl.pallas_call(
    _finish_body,
    in_specs=[
        pl.BlockSpec(memory_space=pltpu.SMEM),
        pl.BlockSpec((_NW, 3 * _NCLS, _SEGP), lambda: (0, 0, 0)),
    ],
    out_specs=pl.BlockSpec(memory_space=pltpu.SMEM),
    out_shape=jax.ShapeDtypeStruct((3,), jnp.float32),
)


def _run(pred, heatmaps, depth, loss_1, pred_depth):
    pred32 = pred.astype(jnp.int32)
    hm32 = heatmaps.astype(jnp.int32)
    d32 = _f64_to_f32(depth)
    l32 = _f64_to_f32(loss_1)
    packed = _pack(pred32, hm32, d32)
    tabs = _sc_scatter(packed.reshape(_N), l32.reshape(_N))
    pred_depth_d = lax.optimization_barrier((pred_depth, packed))[0]
    pd32 = _f64_to_f32(pred_depth_d)
    sums = _sums(d32, pd32, l32, hm32)
    return _finish(sums, tabs.reshape(_NW, 3 * _NCLS, _SEGP))


def kernel(pred, heatmaps, depth, loss_1, pred_depth, epoch):
    odt = jnp.result_type(depth.dtype, jnp.float32)
    with jax.enable_x64(False):
        out = _run(pred, heatmaps, depth, loss_1, pred_depth)
    out = out.astype(odt)
    return out[0], out[1], out[2]


# speedup vs baseline: 2.3090x; 1.0880x over previous
import functools

import jax
import jax.numpy as jnp
from jax import lax
from jax.experimental import pallas as pl
from jax.experimental.pallas import tpu as pltpu
from jax.experimental.pallas import tpu_sc as plsc

_IGNORE = 255
_NCLS = 19
_M = 8
_D = 10
_K = 26
_NSEG = _M * _M * _K
_SEGP = 1792
_TBL_ONE = _NCLS * _SEGP
_TBL = 3 * _TBL_ONE
_B, _W, _H = 4, 512, 512
_N = _B * _W * _H
_NW = 32
_CHUNK = _N // _NW
_SUB = 2048
_NSUB = _CHUNK // _SUB
_MG = 14592
_MROUNDS = _TBL // _MG
_SLICE = _MG // 16


def _f64_to_f32(x):
    return x.astype(jnp.float32)


def _pack_body(pred_ref, hm_ref, depth_ref, packed_ref):
    j = pl.program_id(1)
    rows = lax.broadcasted_iota(jnp.int32, (256, 512), 0) + j * 256
    cols = lax.broadcasted_iota(jnp.int32, (256, 512), 1)
    rb = rows // 64
    cb = cols // 64
    row_in = (rows - rb * 64) < 63
    col_in = (cols - cb * 64) < 63
    blk = rb * _M + cb

    d = depth_ref[0]
    hm = hm_ref[0]
    pr = pred_ref[0]

    kb = jnp.clip(jnp.floor(d / float(_D)).astype(jnp.int32), 0, _K - 1)
    kbf = kb.astype(jnp.float32)
    bin_valid = (d > kbf * _D) & (d < kbf * _D + (_D - 1))
    valid = hm != _IGNORE
    overall = row_in & col_in & valid & bin_valid
    seg = jnp.where(overall, blk * _K + kb, _NSEG)

    hmc = jnp.clip(hm, 0, _NCLS - 1)
    prc = jnp.clip(pr, 0, _NCLS - 1)
    il = hmc * _SEGP + seg
    ip = prc * _SEGP + seg
    packed_ref[0] = il | (ip << 16)


_pack = pl.pallas_call(
    _pack_body,
    grid=(_B, 2),
    in_specs=[
        pl.BlockSpec((1, 256, 512), lambda i, j: (i, j, 0)),
        pl.BlockSpec((1, 256, 512), lambda i, j: (i, j, 0)),
        pl.BlockSpec((1, 256, 512), lambda i, j: (i, j, 0)),
    ],
    out_specs=pl.BlockSpec((1, 256, 512), lambda i, j: (i, j, 0)),
    out_shape=jax.ShapeDtypeStruct((_B, _W, _H), jnp.int32),
)


def _sums_body(depth_ref, pd_ref, loss_ref, hm_ref, sums_ref):
    i = pl.program_id(0)
    j = pl.program_id(1)
    d = depth_ref[0]
    pd = pd_ref[0]
    l1 = loss_ref[0]
    hm = hm_ref[0]

    diff = jnp.log(pd * 255.0 + 1.0) - jnp.log(d + 1.0)
    s0 = jnp.sum(diff * diff)
    s1 = jnp.sum(diff)
    s2 = jnp.sum(l1 * jnp.abs(diff))
    s3 = jnp.sum((hm != _IGNORE).astype(jnp.float32))

    @pl.when((i == 0) & (j == 0))
    def _():
        sums_ref[0] = 0.0
        sums_ref[1] = 0.0
        sums_ref[2] = 0.0
        sums_ref[3] = 0.0

    sums_ref[0] += s0
    sums_ref[1] += s1
    sums_ref[2] += s2
    sums_ref[3] += s3


_sums = pl.pallas_call(
    _sums_body,
    grid=(_B, 2),
    in_specs=[
        pl.BlockSpec((1, 256, 512), lambda i, j: (i, j, 0)),
        pl.BlockSpec((1, 256, 512), lambda i, j: (i, j, 0)),
        pl.BlockSpec((1, 256, 512), lambda i, j: (i, j, 0)),
        pl.BlockSpec((1, 256, 512), lambda i, j: (i, j, 0)),
    ],
    out_specs=pl.BlockSpec(memory_space=pltpu.SMEM, block_shape=(4,), index_map=lambda i, j: (0,)),
    out_shape=jax.ShapeDtypeStruct((4,), jnp.float32),
)


def _sc_body(packed_hbm, loss_hbm, out_hbm, tbl, pk0, pk1, ls0, ls1, sem0, sem1):
    core = lax.axis_index("c")
    sid = lax.axis_index("s")
    wid = core * 16 + sid
    base = wid * _CHUNK

    zf = jnp.zeros((16,), jnp.float32)
    onef = jnp.ones((16,), jnp.float32)
    mask16 = jnp.full((16,), 0xFFFF, jnp.int32)
    shift16 = jnp.full((16,), 16, jnp.int32)
    off_p = jnp.full((16,), _TBL_ONE, jnp.int32)
    off_l = jnp.full((16,), 2 * _TBL_ONE, jnp.int32)

    def zbody(i, c):
        tbl[pl.ds(i * 64, 16)] = zf
        tbl[pl.ds(i * 64 + 16, 16)] = zf
        tbl[pl.ds(i * 64 + 32, 16)] = zf
        tbl[pl.ds(i * 64 + 48, 16)] = zf
        return c

    lax.fori_loop(0, _TBL // 64, zbody, 0, unroll=4)

    bufs = ((pk0, ls0, sem0), (pk1, ls1, sem1))

    def start(k, b):
        pk, ls, sem = bufs[b]
        h0 = pltpu.async_copy(packed_hbm.at[pl.ds(base + k * _SUB, _SUB)], pk, sem)
        h1 = pltpu.async_copy(loss_hbm.at[pl.ds(base + k * _SUB, _SUB)], ls, sem)
        return (h0, h1)

    def process(b):
        pk, ls, _ = bufs[b]

        def pbody(i, c):
            v = pk[pl.ds(i * 16, 16)]
            il = v & mask16
            ip = lax.shift_right_logical(v, shift16)
            lv = ls[pl.ds(i * 16, 16)]
            plsc.addupdate_scatter(tbl, [il], onef)
            plsc.addupdate_scatter(tbl, [ip + off_p], onef)
            plsc.addupdate_scatter(tbl, [il + off_l], lv)
            return c

        lax.fori_loop(0, _SUB // 16, pbody, 0, unroll=4)

    pending = start(0, 0)
    for k in range(_NSUB):
        b = k & 1
        cur = pending
        if k + 1 < _NSUB:
            pending = start(k + 1, 1 - b)
        cur[0].wait()
        cur[1].wait()
        process(b)

    pltpu.sync_copy(tbl, out_hbm.at[pl.ds(wid * _TBL, _TBL)])


_sc_scatter = functools.partial(
    pl.kernel,
    out_type=jax.ShapeDtypeStruct((_NW * _TBL,), jnp.float32),
    mesh=plsc.VectorSubcoreMesh(
        core_axis_name="c", subcore_axis_name="s", num_cores=2, num_subcores=16
    ),
    scratch_types=[
        pltpu.VMEM((_TBL,), jnp.float32),
        pltpu.VMEM((_SUB,), jnp.int32),
        pltpu.VMEM((_SUB,), jnp.int32),
        pltpu.VMEM((_SUB,), jnp.float32),
        pltpu.VMEM((_SUB,), jnp.float32),
        pltpu.SemaphoreType.DMA,
        pltpu.SemaphoreType.DMA,
    ],
    compiler_params=pltpu.CompilerParams(needs_layout_passes=False),
)(_sc_body)


def _finish_body(sums_ref, tab_ref, out_ref):
    t = tab_ref[pl.ds(0, _TBL)]
    for w in range(1, _NW):
        t = t + tab_ref[pl.ds(w * _TBL, _TBL)]

    counts = t[0:_SEGP]
    lsum = t[2 * _TBL_ONE : 2 * _TBL_ONE + _SEGP]
    sym = ((t[0:_SEGP] > 0.0) != (t[_TBL_ONE : _TBL_ONE + _SEGP] > 0.0)).astype(
        jnp.float32
    )
    for c in range(1, _NCLS):
        o = c * _SEGP
        counts = counts + t[o : o + _SEGP]
        lsum = lsum + t[2 * _TBL_ONE + o : 2 * _TBL_ONE + o + _SEGP]
        sym = sym + (
            (t[o : o + _SEGP] > 0.0)
            != (t[_TBL_ONE + o : _TBL_ONE + o + _SEGP] > 0.0)
        ).astype(jnp.float32)

    colid = lax.broadcasted_iota(jnp.int32, (_SEGP,), 0)
    segvalid = colid < _NSEG
    active = (counts > 0.0) & segvalid
    mean = lsum / jnp.maximum(counts, 1.0)
    times = jnp.sum(active.astype(jnp.float32))
    loss3 = jnp.sum(jnp.where(active, sym * mean, 0.0)) / (times + 0.001)

    nf = float(_N)
    data_loss = sums_ref[0] / nf - 0.5 * (sums_ref[1] * sums_ref[1]) / (nf * nf)
    lr = sums_ref[2] / sums_ref[3]
    out_ref[0] = data_loss
    out_ref[1] = lr
    out_ref[2] = loss3


_finish = pl.pallas_call(
    _finish_body,
    in_specs=[
        pl.BlockSpec(memory_space=pltpu.SMEM),
        pl.BlockSpec((_NW * _TBL,), lambda: (0,)),
    ],
    out_specs=pl.BlockSpec(memory_space=pltpu.SMEM),
    out_shape=jax.ShapeDtypeStruct((3,), jnp.float32),
)


def _run(pred, heatmaps, depth, loss_1, pred_depth):
    pred32 = pred.astype(jnp.int32)
    hm32 = heatmaps.astype(jnp.int32)
    d32 = _f64_to_f32(depth)
    l32 = _f64_to_f32(loss_1)
    packed = _pack(pred32, hm32, d32)
    tabs = _sc_scatter(packed.reshape(_N), l32.reshape(_N))
    pred_depth_d = lax.optimization_barrier((pred_depth, packed))[0]
    pd32 = _f64_to_f32(pred_depth_d)
    sums = _sums(d32, pd32, l32, hm32)
    return _finish(sums, tabs)


def kernel(pred, heatmaps, depth, loss_1, pred_depth, epoch):
    odt = jnp.result_type(depth.dtype, jnp.float32)
    with jax.enable_x64(False):
        out = _run(pred, heatmaps, depth, loss_1, pred_depth)
    out = out.astype(odt)
    return out[0], out[1], out[2]


# speedup vs baseline: 2.7504x; 1.1912x over previous
import functools

import jax
import jax.numpy as jnp
from jax import lax
from jax.experimental import pallas as pl
from jax.experimental.pallas import tpu as pltpu
from jax.experimental.pallas import tpu_sc as plsc

_IGNORE = 255
_NCLS = 19
_M = 8
_D = 10
_K = 26
_NSEG = _M * _M * _K
_SEGP = 1792
_TBL_ONE = _NCLS * _SEGP
_TBL = 3 * _TBL_ONE
_B, _W, _H = 4, 512, 512
_N = _B * _W * _H
_NW = 32
_CHUNK = _N // _NW
_SUB = 2048
_NSUB = _CHUNK // _SUB
_MG = 14592
_MROUNDS = _TBL // _MG
_SLICE = _MG // 16


def _f64_to_f32(x):
    return x.astype(jnp.float32)


def _pack_body(pred_ref, hm_ref, depth_ref, packed_ref):
    j = pl.program_id(1)
    rows = lax.broadcasted_iota(jnp.int32, (256, 512), 0) + j * 256
    cols = lax.broadcasted_iota(jnp.int32, (256, 512), 1)
    rb = rows >> 6
    cb = cols >> 6
    row_in = (rows & 63) < 63
    col_in = (cols & 63) < 63
    blk = rb * _M + cb

    d = depth_ref[0]
    hm = hm_ref[0]
    pr = pred_ref[0]

    kb = jnp.clip(jnp.floor(d / float(_D)).astype(jnp.int32), 0, _K - 1)
    kbf = kb.astype(jnp.float32)
    bin_valid = (d > kbf * _D) & (d < kbf * _D + (_D - 1))
    valid = hm != _IGNORE
    overall = row_in & col_in & valid & bin_valid
    seg = jnp.where(overall, blk * _K + kb, _NSEG)

    hmc = jnp.clip(hm, 0, _NCLS - 1)
    prc = jnp.clip(pr, 0, _NCLS - 1)
    il = hmc * _SEGP + seg
    ip = prc * _SEGP + seg
    packed_ref[0] = il | (ip << 16)


_pack = pl.pallas_call(
    _pack_body,
    grid=(_B, 2),
    in_specs=[
        pl.BlockSpec((1, 256, 512), lambda i, j: (i, j, 0)),
        pl.BlockSpec((1, 256, 512), lambda i, j: (i, j, 0)),
        pl.BlockSpec((1, 256, 512), lambda i, j: (i, j, 0)),
    ],
    out_specs=pl.BlockSpec((1, 256, 512), lambda i, j: (i, j, 0)),
    out_shape=jax.ShapeDtypeStruct((_B, _W, _H), jnp.int32),
)


def _sums_body(depth_ref, pd_ref, loss_ref, hm_ref, sums_ref):
    i = pl.program_id(0)
    j = pl.program_id(1)
    d = depth_ref[0]
    pd = pd_ref[0]
    l1 = loss_ref[0]
    hm = hm_ref[0]

    diff = jnp.log(pd * 255.0 + 1.0) - jnp.log(d + 1.0)
    s0 = jnp.sum(diff * diff)
    s1 = jnp.sum(diff)
    s2 = jnp.sum(l1 * jnp.abs(diff))
    s3 = jnp.sum((hm != _IGNORE).astype(jnp.float32))

    @pl.when((i == 0) & (j == 0))
    def _():
        sums_ref[0] = 0.0
        sums_ref[1] = 0.0
        sums_ref[2] = 0.0
        sums_ref[3] = 0.0

    sums_ref[0] += s0
    sums_ref[1] += s1
    sums_ref[2] += s2
    sums_ref[3] += s3


_sums = pl.pallas_call(
    _sums_body,
    grid=(_B, 2),
    in_specs=[
        pl.BlockSpec((1, 256, 512), lambda i, j: (i, j, 0)),
        pl.BlockSpec((1, 256, 512), lambda i, j: (i, j, 0)),
        pl.BlockSpec((1, 256, 512), lambda i, j: (i, j, 0)),
        pl.BlockSpec((1, 256, 512), lambda i, j: (i, j, 0)),
    ],
    out_specs=pl.BlockSpec(memory_space=pltpu.SMEM, block_shape=(4,), index_map=lambda i, j: (0,)),
    out_shape=jax.ShapeDtypeStruct((4,), jnp.float32),
)


def _sc_body(packed_hbm, loss_hbm, out_hbm, tbl, pk0, pk1, ls0, ls1, sem0, sem1):
    core = lax.axis_index("c")
    sid = lax.axis_index("s")
    wid = core * 16 + sid
    base = wid * _CHUNK

    zf = jnp.zeros((16,), jnp.float32)
    onef = jnp.ones((16,), jnp.float32)
    mask16 = jnp.full((16,), 0xFFFF, jnp.int32)
    shift16 = jnp.full((16,), 16, jnp.int32)
    off_p = jnp.full((16,), _TBL_ONE, jnp.int32)
    off_l = jnp.full((16,), 2 * _TBL_ONE, jnp.int32)

    def zbody(i, c):
        tbl[pl.ds(i * 64, 16)] = zf
        tbl[pl.ds(i * 64 + 16, 16)] = zf
        tbl[pl.ds(i * 64 + 32, 16)] = zf
        tbl[pl.ds(i * 64 + 48, 16)] = zf
        return c

    lax.fori_loop(0, _TBL // 64, zbody, 0, unroll=4)

    bufs = ((pk0, ls0, sem0), (pk1, ls1, sem1))

    def start(k, b):
        pk, ls, sem = bufs[b]
        h0 = pltpu.async_copy(packed_hbm.at[pl.ds(base + k * _SUB, _SUB)], pk, sem)
        h1 = pltpu.async_copy(loss_hbm.at[pl.ds(base + k * _SUB, _SUB)], ls, sem)
        return (h0, h1)

    def process(b):
        pk, ls, _ = bufs[b]

        def pbody(i, c):
            v = pk[pl.ds(i * 16, 16)]
            il = v & mask16
            ip = lax.shift_right_logical(v, shift16)
            lv = ls[pl.ds(i * 16, 16)]
            plsc.addupdate_scatter(tbl, [il], onef)
            plsc.addupdate_scatter(tbl, [ip + off_p], onef)
            plsc.addupdate_scatter(tbl, [il + off_l], lv)
            return c

        lax.fori_loop(0, _SUB // 16, pbody, 0, unroll=4)

    pending = start(0, 0)
    for k in range(_NSUB):
        b = k & 1
        cur = pending
        if k + 1 < _NSUB:
            pending = start(k + 1, 1 - b)
        cur[0].wait()
        cur[1].wait()
        process(b)

    pltpu.sync_copy(tbl, out_hbm.at[pl.ds(wid * _TBL, _TBL)])


_sc_scatter = functools.partial(
    pl.kernel,
    out_type=jax.ShapeDtypeStruct((_NW * _TBL,), jnp.float32),
    mesh=plsc.VectorSubcoreMesh(
        core_axis_name="c", subcore_axis_name="s", num_cores=2, num_subcores=16
    ),
    scratch_types=[
        pltpu.VMEM((_TBL,), jnp.float32),
        pltpu.VMEM((_SUB,), jnp.int32),
        pltpu.VMEM((_SUB,), jnp.int32),
        pltpu.VMEM((_SUB,), jnp.float32),
        pltpu.VMEM((_SUB,), jnp.float32),
        pltpu.SemaphoreType.DMA,
        pltpu.SemaphoreType.DMA,
    ],
    compiler_params=pltpu.CompilerParams(needs_layout_passes=False),
)(_sc_body)


def _finish_body(sums_ref, tab_ref, out_ref):
    t = tab_ref[pl.ds(0, _TBL)]
    for w in range(1, _NW):
        t = t + tab_ref[pl.ds(w * _TBL, _TBL)]

    counts = t[0:_SEGP]
    lsum = t[2 * _TBL_ONE : 2 * _TBL_ONE + _SEGP]
    sym = ((t[0:_SEGP] > 0.0) != (t[_TBL_ONE : _TBL_ONE + _SEGP] > 0.0)).astype(
        jnp.float32
    )
    for c in range(1, _NCLS):
        o = c * _SEGP
        counts = counts + t[o : o + _SEGP]
        lsum = lsum + t[2 * _TBL_ONE + o : 2 * _TBL_ONE + o + _SEGP]
        sym = sym + (
            (t[o : o + _SEGP] > 0.0)
            != (t[_TBL_ONE + o : _TBL_ONE + o + _SEGP] > 0.0)
        ).astype(jnp.float32)

    colid = lax.broadcasted_iota(jnp.int32, (_SEGP,), 0)
    segvalid = colid < _NSEG
    active = (counts > 0.0) & segvalid
    mean = lsum / jnp.maximum(counts, 1.0)
    times = jnp.sum(active.astype(jnp.float32))
    loss3 = jnp.sum(jnp.where(active, sym * mean, 0.0)) / (times + 0.001)

    nf = float(_N)
    data_loss = sums_ref[0] / nf - 0.5 * (sums_ref[1] * sums_ref[1]) / (nf * nf)
    lr = sums_ref[2] / sums_ref[3]
    out_ref[0] = data_loss
    out_ref[1] = lr
    out_ref[2] = loss3


_finish = pl.pallas_call(
    _finish_body,
    in_specs=[
        pl.BlockSpec(memory_space=pltpu.SMEM),
        pl.BlockSpec((_NW * _TBL,), lambda: (0,)),
    ],
    out_specs=pl.BlockSpec(memory_space=pltpu.SMEM),
    out_shape=jax.ShapeDtypeStruct((3,), jnp.float32),
)


def _run(pred, heatmaps, depth, loss_1, pred_depth):
    pred32 = pred.astype(jnp.int32)
    hm32 = heatmaps.astype(jnp.int32)
    d32 = _f64_to_f32(depth)
    l32 = _f64_to_f32(loss_1)
    d1d = lax.optimization_barrier((d32.reshape(_N),))[0]
    d3 = d1d.reshape(_B, _W, _H)
    packed = _pack(pred32, hm32, d3)
    tabs = _sc_scatter(packed.reshape(_N), l32.reshape(_N))
    pred_depth_d = lax.optimization_barrier((pred_depth, packed))[0]
    pd32 = _f64_to_f32(pred_depth_d)
    sums = _sums(d32, pd32, l32, hm32)
    return _finish(sums, tabs)


def kernel(pred, heatmaps, depth, loss_1, pred_depth, epoch):
    odt = jnp.result_type(depth.dtype, jnp.float32)
    with jax.enable_x64(False):
        out = _run(pred, heatmaps, depth, loss_1, pred_depth)
    out = out.astype(odt)
    return out[0], out[1], out[2]
